# baseline (speedup 1.0000x reference)
import jax
import jax.numpy as jnp
from jax import lax
from jax.experimental import pallas as pl
from jax.experimental.pallas import tpu as pltpu

_ROWS = 16384
_COLS = 512
_CHR = 512
_ECHR = 128
_NE = 4
_NBUF = 4
_MAIN0 = _NE * _ECHR
_NMAIN = (_ROWS - 2 * _NE * _ECHR) // _CHR
_END0 = _MAIN0 + _NMAIN * _CHR


def _bce_block(x, t):
    a = jnp.abs(x)
    sp = jnp.log(1.0 + jnp.exp(-a))
    loss = 0.5 * a - 0.125 * (x * t) + sp
    return jnp.where(t > 0.0, loss, 0.0)


def _tc_body(p_hbm, t_hbm, o_ref, pbuf, tbuf, pse, tse,
             psem, tsem, esem_p, esem_t):
    def m_copy(hbm, buf, sem, ci, slot):
        return pltpu.make_async_copy(
            hbm.at[pl.ds(_MAIN0 + ci * _CHR, _CHR), :],
            buf.at[slot], sem.at[slot])

    def e_copy(hbm, buf, sem, row0, k):
        return pltpu.make_async_copy(
            hbm.at[pl.ds(row0 + k * _ECHR, _ECHR), :],
            buf.at[k], sem.at[k])

    for k in range(_NE):
        e_copy(p_hbm, pse, esem_p, 0, k).start()
        e_copy(t_hbm, tse, esem_t, 0, k).start()
    for ci in range(_NBUF):
        m_copy(p_hbm, pbuf, psem, ci, ci).start()
        m_copy(t_hbm, tbuf, tsem, ci, ci).start()

    o_ref[0] = 0.0

    for k in range(_NE):
        e_copy(p_hbm, pse, esem_p, 0, k).wait()
        e_copy(t_hbm, tse, esem_t, 0, k).wait()
        o_ref[0] += jnp.sum(_bce_block(pse[k], tse[k]))

    def chunk(ci, carry):
        slot = lax.rem(ci, _NBUF)
        m_copy(p_hbm, pbuf, psem, ci, slot).wait()
        m_copy(t_hbm, tbuf, tsem, ci, slot).wait()
        o_ref[0] += jnp.sum(_bce_block(pbuf[slot], tbuf[slot]))

        @pl.when(ci + _NBUF < _NMAIN)
        def _prefetch():
            m_copy(p_hbm, pbuf, psem, ci + _NBUF, slot).start()
            m_copy(t_hbm, tbuf, tsem, ci + _NBUF, slot).start()

        for k in range(_NE):
            @pl.when(ci == _NMAIN - _NE + k)
            def _tail(k=k):
                e_copy(p_hbm, pse, esem_p, _END0, k).start()
                e_copy(t_hbm, tse, esem_t, _END0, k).start()

        return carry

    lax.fori_loop(0, _NMAIN, chunk, 0)

    for k in range(_NE):
        e_copy(p_hbm, pse, esem_p, _END0, k).wait()
        e_copy(t_hbm, tse, esem_t, _END0, k).wait()
        o_ref[0] += jnp.sum(_bce_block(pse[k], tse[k]))


def kernel(pred_map, target_map):
    p = pred_map.reshape(_ROWS, _COLS)
    t = target_map.reshape(_ROWS, _COLS)
    out = pl.pallas_call(
        _tc_body,
        in_specs=[
            pl.BlockSpec(memory_space=pl.ANY),
            pl.BlockSpec(memory_space=pl.ANY),
        ],
        out_specs=pl.BlockSpec(memory_space=pltpu.SMEM),
        out_shape=jax.ShapeDtypeStruct((1,), jnp.float32),
        scratch_shapes=[
            pltpu.VMEM((_NBUF, _CHR, _COLS), jnp.float32),
            pltpu.VMEM((_NBUF, _CHR, _COLS), jnp.float32),
            pltpu.VMEM((_NE, _ECHR, _COLS), jnp.float32),
            pltpu.VMEM((_NE, _ECHR, _COLS), jnp.float32),
            pltpu.SemaphoreType.DMA((_NBUF,)),
            pltpu.SemaphoreType.DMA((_NBUF,)),
            pltpu.SemaphoreType.DMA((_NE,)),
            pltpu.SemaphoreType.DMA((_NE,)),
        ],
    )(p, t)
    return out[0]
